# baseline (speedup 1.0000x reference)
import functools

import jax
import jax.numpy as jnp
from jax import lax
from jax.experimental import pallas as pl
from jax.experimental.pallas import tpu as pltpu
from jax.experimental.pallas import tpu_sc as plsc

NC = 2
NS = 16
NW = NC * NS
NBLK = 32
NBS = 3
L = 16

def kernel(phone, table):
    B, S = phone.shape
    V, D = table.shape
    per_w = B // NW
    n_blk = per_w // NBLK
    n_full = S // L
    tail = S - n_full * L
    idx3 = phone.reshape(NW, per_w, S)
    table_p = jnp.pad(table, ((0, 0), (0, 128 - D)))

    mesh = plsc.VectorSubcoreMesh(core_axis_name="c", subcore_axis_name="s")

    @functools.partial(
        pl.kernel,
        mesh=mesh,
        out_type=jax.ShapeDtypeStruct((B, S, D), jnp.float32),
        compiler_params=pltpu.CompilerParams(needs_layout_passes=False),
        scratch_types=[
            pltpu.VMEM((V, 128), jnp.float32),
            pltpu.VMEM((2, NBLK, S), jnp.int32),
            pltpu.VMEM((NBS, S, D), jnp.float32),
            pltpu.SemaphoreType.DMA((2,)),
            pltpu.SemaphoreType.DMA((NBS,)),
        ],
    )
    def emb(idx_hbm, table_hbm, out_hbm, tbl_v, ibuf, cbuf, isem, ssem):
        wid = lax.axis_index("s") * NC + lax.axis_index("c")
        base = wid * per_w

        def idx_block(m):
            bm = m % 2
            return (
                idx_hbm.at[wid, pl.ds(m * NBLK, NBLK)],
                ibuf.at[bm],
                isem.at[bm],
            )

        pltpu.async_copy(*idx_block(0))
        pltpu.sync_copy(table_hbm, tbl_v)
        iota = lax.iota(jnp.int32, L)
        cols = [g * L + iota for g in range(D // L)]

        for m in range(n_blk):
            if m + 1 < n_blk:
                pltpu.async_copy(*idx_block(m + 1))
            pltpu.make_async_copy(*idx_block(m)).wait()
            bm = m % 2

            def body(jj, carry):
                j = m * NBLK + jj
                bs = lax.rem(j, NBS)

                @pl.when(j >= NBS)
                def _():
                    pltpu.make_async_copy(
                        cbuf.at[bs], out_hbm.at[base + j - NBS], ssem.at[bs]
                    ).wait()

                def group(start, nrows):
                    slab = cbuf.at[bs]
                    v_bm = jnp.full((L,), bm, jnp.int32)
                    v_jj = jnp.full((L,), jj, jnp.int32)
                    for l in range(nrows):
                        row = plsc.load_gather(
                            ibuf,
                            [v_bm, v_jj, jnp.full((L,), start + l, jnp.int32)],
                        )
                        out_row = jnp.full((L,), start + l, jnp.int32)
                        for g, col in enumerate(cols):
                            val = plsc.load_gather(tbl_v, [row, col])
                            plsc.store_scatter(slab, [out_row, col], val)

                def fullg(gg, c):
                    group(gg * L, L)
                    return c

                lax.fori_loop(0, n_full, fullg, 0)
                if tail:
                    group(S - L, L)

                pltpu.async_copy(
                    cbuf.at[bs], out_hbm.at[base + j], ssem.at[bs]
                )
                return carry

            lax.fori_loop(0, NBLK, body, 0)

        for i in range(NBS):
            j = per_w - NBS + i
            pltpu.make_async_copy(
                cbuf.at[j % NBS], out_hbm.at[base + j], ssem.at[j % NBS]
            ).wait()

    return emb(idx3, table_p)
